# baseline (speedup 1.0000x reference)
import functools

import jax
import jax.numpy as jnp
from jax.experimental import pallas as pl
from jax.experimental.pallas import tpu as pltpu


def _round_up(x: int, m: int) -> int:
    return ((x + m - 1) // m) * m


def _make_main_kernel(tile_b: int, chunk: int):
    n_chunks = tile_b // chunk

    def _main(uidx_ref, iidx_ref,
              ut_ref, it_ref,
              w1u_ref, w1i_ref, b1_ref,
              w2_ref, b2_ref,
              w3_ref, b3_ref,
              wfg_ref, wfm_ref,
              bf_ref,
              out_ref,
              *scratch):
        f32 = jnp.float32
        for c in range(n_chunks):
            au_ref = scratch[2 * c]
            ai_ref = scratch[2 * c + 1]
            base = c * chunk
            for r in range(chunk):
                au_ref[r, :] = ut_ref[uidx_ref[0, 0, base + r], 0]
                ai_ref[r, :] = it_ref[iidx_ref[0, 0, base + r], 0]
            u = au_ref[...]
            it = ai_ref[...]
            h = jnp.maximum(
                jnp.dot(u, w1u_ref[...], preferred_element_type=f32)
                + jnp.dot(it, w1i_ref[...], preferred_element_type=f32)
                + b1_ref[...], 0.0)
            h = jnp.maximum(
                jnp.dot(h, w2_ref[...], preferred_element_type=f32)
                + b2_ref[...], 0.0)
            h = jnp.maximum(
                jnp.dot(h, w3_ref[...], preferred_element_type=f32)
                + b3_ref[...], 0.0)
            g = u * it
            score = (jnp.dot(g, wfg_ref[...], preferred_element_type=f32)
                     + jnp.dot(h, wfm_ref[...], preferred_element_type=f32)
                     + bf_ref[...])
            out_ref[pl.ds(base, chunk), :] = jax.nn.sigmoid(score)
    return _main


@functools.partial(jax.jit, static_argnames=("tile_b", "chunk"))
def _forward(user_idx, item_idx, user_emb, item_emb,
             w1, b1, w2, b2, w3, b3, wf, bf, *,
             tile_b: int = 4096, chunk: int = 512):
    B = int(user_idx.shape[0])
    U, W = user_emb.shape
    I = item_emb.shape[0]
    half = w1.shape[0] // 2
    mf_dim = W - half
    l1 = w1.shape[1]

    ut3 = user_emb.reshape(U, 1, W)
    it3 = item_emb.reshape(I, 1, W)

    zeros_mf = jnp.zeros((mf_dim, l1), jnp.float32)
    w1u_pad = jnp.concatenate([zeros_mf, w1[:half, :]], axis=0)
    w1i_pad = jnp.concatenate([zeros_mf, w1[half:, :]], axis=0)
    wfg_col = jnp.concatenate([wf[:mf_dim, :],
                               jnp.zeros((half, 1), jnp.float32)], axis=0)
    wfm_col = wf[mf_dim:, :]

    b_pad = _round_up(B, tile_b)
    pad = b_pad - B
    uidx = jnp.pad(user_idx.astype(jnp.int32), (0, pad)).reshape(-1, 1, tile_b)
    iidx = jnp.pad(item_idx.astype(jnp.int32), (0, pad)).reshape(-1, 1, tile_b)
    num_tiles = b_pad // tile_b

    idx_spec = pl.BlockSpec((1, 1, tile_b), lambda i: (i, 0, 0),
                            memory_space=pltpu.SMEM)

    def _whole(a):
        return pl.BlockSpec(a.shape, lambda i: (0,) * a.ndim)

    out = pl.pallas_call(
        _make_main_kernel(tile_b, chunk),
        out_shape=jax.ShapeDtypeStruct((b_pad, 1), jnp.float32),
        grid=(num_tiles,),
        in_specs=[idx_spec, idx_spec,
                  _whole(ut3), _whole(it3),
                  _whole(w1u_pad), _whole(w1i_pad), _whole(b1),
                  _whole(w2), _whole(b2), _whole(w3), _whole(b3),
                  _whole(wfg_col), _whole(wfm_col), _whole(bf)],
        out_specs=pl.BlockSpec((tile_b, 1), lambda i: (i, 0)),
        scratch_shapes=[pltpu.VMEM((chunk, W), jnp.float32)
                        for _ in range(2 * (tile_b // chunk))],
        compiler_params=pltpu.CompilerParams(
            dimension_semantics=("parallel",),
            vmem_limit_bytes=64 * 1024 * 1024,
        ),
    )(uidx, iidx, ut3, it3, w1u_pad, w1i_pad, b1,
      w2, b2, w3, b3, wfg_col, wfm_col, bf)
    return out[:B]


def kernel(user_idx, item_idx, user_emb, item_emb, w1, b1, w2, b2, w3, b3, wf, bf):
    return _forward(user_idx, item_idx, user_emb, item_emb,
                    w1, b1, w2, b2, w3, b3, wf, bf)


# speedup vs baseline: 1.4389x; 1.4389x over previous
import functools

import jax
import jax.numpy as jnp
from jax.experimental import pallas as pl
from jax.experimental.pallas import tpu as pltpu


def _round_up(x: int, m: int) -> int:
    return ((x + m - 1) // m) * m


def _make_transform_kernel(mf_dim: int):
    def _transform(tabT_ref, w_ref, b_ref, d_ref, out_ref):
        f32 = jnp.float32
        dn = (((0,), (0,)), ((), ()))
        gm = jax.lax.dot_general(tabT_ref[:mf_dim, :], d_ref[...], dn,
                                 preferred_element_type=f32)
        ml = jax.lax.dot_general(tabT_ref[mf_dim:, :], w_ref[...], dn,
                                 preferred_element_type=f32) + b_ref[...]
        out_ref[...] = jnp.concatenate([gm, ml], axis=1)
    return _transform


def _transform_table(tabT, w, b_row, diag, *, mf_dim: int,
                     row_tile: int = 1024):
    width, n = tabT.shape
    n_pad = _round_up(n, row_tile)
    tabT_p = jnp.pad(tabT, ((0, 0), (0, n_pad - n)))
    out = pl.pallas_call(
        _make_transform_kernel(mf_dim),
        out_shape=jax.ShapeDtypeStruct((n_pad, width), jnp.float32),
        grid=(n_pad // row_tile,),
        in_specs=[
            pl.BlockSpec((width, row_tile), lambda i: (0, i)),
            pl.BlockSpec(w.shape, lambda i: (0, 0)),
            pl.BlockSpec(b_row.shape, lambda i: (0, 0)),
            pl.BlockSpec(diag.shape, lambda i: (0, 0)),
        ],
        out_specs=pl.BlockSpec((row_tile, width), lambda i: (i, 0)),
        compiler_params=pltpu.CompilerParams(
            dimension_semantics=("parallel",),
            fuse_transposed_lhs_in_matmul=True),
    )(tabT_p, w, b_row, diag)
    return out


def _make_main_kernel(tile_b: int, chunk: int):
    n_chunks = tile_b // chunk

    def _main(uidx_ref, iidx_ref,
              tu_ref, ti_ref,
              w2p_ref, b2_ref,
              w3p_ref, b3p_ref,
              wfin_ref,
              bf_ref,
              out_ref,
              *scratch):
        f32 = jnp.float32
        for c in range(n_chunks):
            au_ref = scratch[2 * c]
            ai_ref = scratch[2 * c + 1]
            base = c * chunk
            for r in range(chunk):
                au_ref[r, :] = tu_ref[uidx_ref[0, 0, base + r], 0]
                ai_ref[r, :] = ti_ref[iidx_ref[0, 0, base + r], 0]
            a = au_ref[...]
            b = ai_ref[...]
            h = jnp.maximum(a + b, 0.0)
            h = jnp.maximum(
                jnp.dot(h, w2p_ref[...], preferred_element_type=f32)
                + b2_ref[...], 0.0)
            h = jnp.maximum(
                jnp.dot(h, w3p_ref[...], preferred_element_type=f32)
                + b3p_ref[...], 0.0)
            s = h + jnp.pad(a * b, ((0, 0), (0, h.shape[1] - a.shape[1])))
            dn = (((0,), (1,)), ((), ()))
            score_row = jax.lax.dot_general(
                wfin_ref[...], s, dn, preferred_element_type=f32) + bf_ref[...]
            out_ref[0, 0, pl.ds(base, chunk)] = jax.nn.sigmoid(
                score_row)[0, :]
    return _main


@functools.partial(jax.jit, static_argnames=("tile_b", "chunk"))
def _forward(user_idx, item_idx, user_emb, item_emb,
             w1, b1, w2, b2, w3, b3, wf, bf, *,
             tile_b: int = 4096, chunk: int = 512):
    B = int(user_idx.shape[0])
    U, W = user_emb.shape
    half = w1.shape[0] // 2
    mf_dim = W - half
    l2 = w2.shape[1]

    diag_u = jnp.diag(wf[:mf_dim, 0])
    diag_i = jnp.eye(mf_dim, dtype=jnp.float32)
    zero_b = jnp.zeros_like(b1)
    tu = _transform_table(user_emb.T, w1[:half, :], zero_b, diag_u,
                          mf_dim=mf_dim)
    ti = _transform_table(item_emb.T, w1[half:, :], b1, diag_i,
                          mf_dim=mf_dim)
    tu3 = tu.reshape(tu.shape[0], 1, W)
    ti3 = ti.reshape(ti.shape[0], 1, W)

    w2p = jnp.concatenate([jnp.zeros((mf_dim, l2), jnp.float32), w2], axis=0)
    w3p = jnp.pad(w3, ((0, 0), (W, 0)))
    b3p = jnp.pad(b3, ((0, 0), (W, 0)))
    wfin = jnp.concatenate([wf[:mf_dim, :],
                            jnp.zeros((half, 1), jnp.float32),
                            wf[mf_dim:, :]], axis=0)

    b_pad = _round_up(B, tile_b)
    pad = b_pad - B
    uidx = jnp.pad(user_idx.astype(jnp.int32), (0, pad)).reshape(-1, 1, tile_b)
    iidx = jnp.pad(item_idx.astype(jnp.int32), (0, pad)).reshape(-1, 1, tile_b)
    num_tiles = b_pad // tile_b

    idx_spec = pl.BlockSpec((1, 1, tile_b), lambda i: (i, 0, 0),
                            memory_space=pltpu.SMEM)

    def _whole(a):
        return pl.BlockSpec(a.shape, lambda i: (0,) * a.ndim)

    out = pl.pallas_call(
        _make_main_kernel(tile_b, chunk),
        out_shape=jax.ShapeDtypeStruct((num_tiles, 1, tile_b), jnp.float32),
        grid=(num_tiles,),
        in_specs=[idx_spec, idx_spec,
                  _whole(tu3), _whole(ti3),
                  _whole(w2p), _whole(b2), _whole(w3p), _whole(b3p),
                  _whole(wfin), _whole(bf)],
        out_specs=pl.BlockSpec((1, 1, tile_b), lambda i: (i, 0, 0)),
        scratch_shapes=[pltpu.VMEM((chunk, W), jnp.float32)
                        for _ in range(2 * (tile_b // chunk))],
        compiler_params=pltpu.CompilerParams(
            dimension_semantics=("parallel",),
            vmem_limit_bytes=64 * 1024 * 1024,
        ),
    )(uidx, iidx, tu3, ti3, w2p, b2, w3p, b3p, wfin, bf)
    return out.reshape(b_pad)[:B].reshape(B, 1)


def kernel(user_idx, item_idx, user_emb, item_emb, w1, b1, w2, b2, w3, b3, wf, bf):
    return _forward(user_idx, item_idx, user_emb, item_emb,
                    w1, b1, w2, b2, w3, b3, wf, bf)


# speedup vs baseline: 1.4707x; 1.0222x over previous
import functools

import jax
import jax.numpy as jnp
from jax.experimental import pallas as pl
from jax.experimental.pallas import tpu as pltpu


def _round_up(x: int, m: int) -> int:
    return ((x + m - 1) // m) * m


def _make_transform_kernel(mf_dim: int):
    def _transform(tabT_ref, w_ref, b_ref, d_ref, out_ref):
        f32 = jnp.float32
        dn = (((0,), (0,)), ((), ()))
        gm = jax.lax.dot_general(tabT_ref[:mf_dim, :], d_ref[...], dn,
                                 preferred_element_type=f32)
        ml = jax.lax.dot_general(tabT_ref[mf_dim:, :], w_ref[...], dn,
                                 preferred_element_type=f32) + b_ref[...]
        out_ref[...] = jnp.concatenate([gm, ml], axis=1)
    return _transform


def _transform_table(tabT, w, b_row, diag, *, mf_dim: int,
                     row_tile: int = 1024):
    width, n = tabT.shape
    n_pad = _round_up(n, row_tile)
    tabT_p = jnp.pad(tabT, ((0, 0), (0, n_pad - n)))
    out = pl.pallas_call(
        _make_transform_kernel(mf_dim),
        out_shape=jax.ShapeDtypeStruct((n_pad, width), jnp.float32),
        grid=(n_pad // row_tile,),
        in_specs=[
            pl.BlockSpec((width, row_tile), lambda i: (0, i)),
            pl.BlockSpec(w.shape, lambda i: (0, 0)),
            pl.BlockSpec(b_row.shape, lambda i: (0, 0)),
            pl.BlockSpec(diag.shape, lambda i: (0, 0)),
        ],
        out_specs=pl.BlockSpec((row_tile, width), lambda i: (i, 0)),
        compiler_params=pltpu.CompilerParams(
            dimension_semantics=("parallel",),
            fuse_transposed_lhs_in_matmul=True),
    )(tabT_p, w, b_row, diag)
    return out


def _make_main_kernel(tile_b: int, chunk: int):
    n_chunks = tile_b // chunk

    def _main(uidx_ref, iidx_ref,
              tu_ref, ti_ref,
              w2p_ref, b2_ref,
              w3p_ref, b3p_ref,
              wfin_ref,
              bf_ref,
              out_ref,
              *scratch):
        f32 = jnp.float32
        for c in range(n_chunks):
            au_ref = scratch[2 * c]
            ai_ref = scratch[2 * c + 1]
            base = c * chunk
            for r in range(chunk):
                au_ref[r, :] = tu_ref[uidx_ref[0, 0, base + r], 0]
                ai_ref[r, :] = ti_ref[iidx_ref[0, 0, base + r], 0]
            a = au_ref[...]
            b = ai_ref[...]
            h = jnp.maximum(a + b, 0.0)
            h = jnp.maximum(
                jnp.dot(h, w2p_ref[...], preferred_element_type=f32)
                + b2_ref[...], 0.0)
            h = jnp.maximum(
                jnp.dot(h, w3p_ref[...], preferred_element_type=f32)
                + b3p_ref[...], 0.0)
            s = h + jnp.pad(a * b, ((0, 0), (0, h.shape[1] - a.shape[1])))
            dn = (((0,), (1,)), ((), ()))
            score_row = jax.lax.dot_general(
                wfin_ref[...], s, dn, preferred_element_type=f32) + bf_ref[...]
            out_ref[0, 0, pl.ds(base, chunk)] = jax.nn.sigmoid(
                score_row)[0, :]
    return _main


@functools.partial(jax.jit, static_argnames=("tile_b", "chunk"))
def _forward(user_idx, item_idx, user_emb, item_emb,
             w1, b1, w2, b2, w3, b3, wf, bf, *,
             tile_b: int = 8192, chunk: int = 512):
    B = int(user_idx.shape[0])
    U, W = user_emb.shape
    half = w1.shape[0] // 2
    mf_dim = W - half
    l2 = w2.shape[1]

    diag_u = jnp.diag(wf[:mf_dim, 0])
    diag_i = jnp.eye(mf_dim, dtype=jnp.float32)
    zero_b = jnp.zeros_like(b1)
    tu = _transform_table(user_emb.T, w1[:half, :], zero_b, diag_u,
                          mf_dim=mf_dim)
    ti = _transform_table(item_emb.T, w1[half:, :], b1, diag_i,
                          mf_dim=mf_dim)
    tu3 = tu.reshape(tu.shape[0], 1, W)
    ti3 = ti.reshape(ti.shape[0], 1, W)

    w2p = jnp.concatenate([jnp.zeros((mf_dim, l2), jnp.float32), w2], axis=0)
    w3p = jnp.pad(w3, ((0, 0), (W, 0)))
    b3p = jnp.pad(b3, ((0, 0), (W, 0)))
    wfin = jnp.concatenate([wf[:mf_dim, :],
                            jnp.zeros((half, 1), jnp.float32),
                            wf[mf_dim:, :]], axis=0)

    b_pad = _round_up(B, tile_b)
    pad = b_pad - B
    uidx = jnp.pad(user_idx.astype(jnp.int32), (0, pad)).reshape(-1, 1, tile_b)
    iidx = jnp.pad(item_idx.astype(jnp.int32), (0, pad)).reshape(-1, 1, tile_b)
    num_tiles = b_pad // tile_b

    idx_spec = pl.BlockSpec((1, 1, tile_b), lambda i: (i, 0, 0),
                            memory_space=pltpu.SMEM)

    def _whole(a):
        return pl.BlockSpec(a.shape, lambda i: (0,) * a.ndim)

    out = pl.pallas_call(
        _make_main_kernel(tile_b, chunk),
        out_shape=jax.ShapeDtypeStruct((num_tiles, 1, tile_b), jnp.float32),
        grid=(num_tiles,),
        in_specs=[idx_spec, idx_spec,
                  _whole(tu3), _whole(ti3),
                  _whole(w2p), _whole(b2), _whole(w3p), _whole(b3p),
                  _whole(wfin), _whole(bf)],
        out_specs=pl.BlockSpec((1, 1, tile_b), lambda i: (i, 0, 0)),
        scratch_shapes=[pltpu.VMEM((chunk, W), jnp.float32)
                        for _ in range(2 * (tile_b // chunk))],
        compiler_params=pltpu.CompilerParams(
            dimension_semantics=("parallel",),
            vmem_limit_bytes=64 * 1024 * 1024,
        ),
    )(uidx, iidx, tu3, ti3, w2p, b2, w3p, b3p, wfin, bf)
    return out.reshape(b_pad)[:B].reshape(B, 1)


def kernel(user_idx, item_idx, user_emb, item_emb, w1, b1, w2, b2, w3, b3, wf, bf):
    return _forward(user_idx, item_idx, user_emb, item_emb,
                    w1, b1, w2, b2, w3, b3, wf, bf)


# speedup vs baseline: 1.4873x; 1.0112x over previous
import functools

import jax
import jax.numpy as jnp
from jax.experimental import pallas as pl
from jax.experimental.pallas import tpu as pltpu


def _round_up(x: int, m: int) -> int:
    return ((x + m - 1) // m) * m


def _make_transform_kernel(mf_dim: int):
    def _transform(tabT_ref, w_ref, b_ref, d_ref, out_ref):
        f32 = jnp.float32
        dn = (((0,), (0,)), ((), ()))
        gm = jax.lax.dot_general(tabT_ref[:mf_dim, :], d_ref[...], dn,
                                 preferred_element_type=f32)
        ml = jax.lax.dot_general(tabT_ref[mf_dim:, :], w_ref[...], dn,
                                 preferred_element_type=f32) + b_ref[...]
        out_ref[...] = jnp.concatenate([gm, ml], axis=1)
    return _transform


def _transform_table(tabT, w, b_row, diag, *, mf_dim: int,
                     row_tile: int = 2048):
    width, n = tabT.shape
    n_pad = _round_up(n, row_tile)
    tabT_p = jnp.pad(tabT, ((0, 0), (0, n_pad - n)))
    out = pl.pallas_call(
        _make_transform_kernel(mf_dim),
        out_shape=jax.ShapeDtypeStruct((n_pad, width), jnp.float32),
        grid=(n_pad // row_tile,),
        in_specs=[
            pl.BlockSpec((width, row_tile), lambda i: (0, i)),
            pl.BlockSpec(w.shape, lambda i: (0, 0)),
            pl.BlockSpec(b_row.shape, lambda i: (0, 0)),
            pl.BlockSpec(diag.shape, lambda i: (0, 0)),
        ],
        out_specs=pl.BlockSpec((row_tile, width), lambda i: (i, 0)),
        compiler_params=pltpu.CompilerParams(
            dimension_semantics=("parallel",),
            fuse_transposed_lhs_in_matmul=True),
    )(tabT_p, w, b_row, diag)
    return out


def _make_main_kernel(tile_b: int, chunk: int):
    n_chunks = tile_b // chunk

    def _main(uidx_ref, iidx_ref,
              tu_ref, ti_ref,
              w2p_ref, b2_ref,
              w3p_ref, b3p_ref,
              wfin_ref,
              bf_ref,
              out_ref,
              *scratch):
        f32 = jnp.float32
        for c in range(n_chunks):
            au_ref = scratch[2 * c]
            ai_ref = scratch[2 * c + 1]
            base = c * chunk
            for r in range(chunk):
                au_ref[r, :] = tu_ref[uidx_ref[0, 0, base + r], 0]
                ai_ref[r, :] = ti_ref[iidx_ref[0, 0, base + r], 0]
            a = au_ref[...]
            b = ai_ref[...]
            h = jnp.maximum(a + b, 0.0)
            h = jnp.maximum(
                jnp.dot(h, w2p_ref[...], preferred_element_type=f32)
                + b2_ref[...], 0.0)
            h = jnp.maximum(
                jnp.dot(h, w3p_ref[...], preferred_element_type=f32)
                + b3p_ref[...], 0.0)
            s = h + jnp.pad(a * b, ((0, 0), (0, h.shape[1] - a.shape[1])))
            dn = (((0,), (1,)), ((), ()))
            score_row = jax.lax.dot_general(
                wfin_ref[...], s, dn, preferred_element_type=f32) + bf_ref[...]
            out_ref[0, 0, pl.ds(base, chunk)] = jax.nn.sigmoid(
                score_row)[0, :]
    return _main


@functools.partial(jax.jit, static_argnames=("tile_b", "chunk"))
def _forward(user_idx, item_idx, user_emb, item_emb,
             w1, b1, w2, b2, w3, b3, wf, bf, *,
             tile_b: int = 8192, chunk: int = 512):
    B = int(user_idx.shape[0])
    U, W = user_emb.shape
    half = w1.shape[0] // 2
    mf_dim = W - half
    l2 = w2.shape[1]

    diag_u = jnp.diag(wf[:mf_dim, 0])
    diag_i = jnp.eye(mf_dim, dtype=jnp.float32)
    zero_b = jnp.zeros_like(b1)
    tu = _transform_table(user_emb.T, w1[:half, :], zero_b, diag_u,
                          mf_dim=mf_dim)
    ti = _transform_table(item_emb.T, w1[half:, :], b1, diag_i,
                          mf_dim=mf_dim)
    tu3 = tu.reshape(tu.shape[0], 1, W)
    ti3 = ti.reshape(ti.shape[0], 1, W)

    w2p = jnp.concatenate([jnp.zeros((mf_dim, l2), jnp.float32), w2], axis=0)
    w3p = jnp.pad(w3, ((0, 0), (W, 0)))
    b3p = jnp.pad(b3, ((0, 0), (W, 0)))
    wfin = jnp.concatenate([wf[:mf_dim, :],
                            jnp.zeros((half, 1), jnp.float32),
                            wf[mf_dim:, :]], axis=0)

    b_pad = _round_up(B, tile_b)
    pad = b_pad - B
    uidx = jnp.pad(user_idx.astype(jnp.int32), (0, pad)).reshape(-1, 1, tile_b)
    iidx = jnp.pad(item_idx.astype(jnp.int32), (0, pad)).reshape(-1, 1, tile_b)
    num_tiles = b_pad // tile_b

    idx_spec = pl.BlockSpec((1, 1, tile_b), lambda i: (i, 0, 0),
                            memory_space=pltpu.SMEM)

    def _whole(a):
        return pl.BlockSpec(a.shape, lambda i: (0,) * a.ndim)

    out = pl.pallas_call(
        _make_main_kernel(tile_b, chunk),
        out_shape=jax.ShapeDtypeStruct((num_tiles, 1, tile_b), jnp.float32),
        grid=(num_tiles,),
        in_specs=[idx_spec, idx_spec,
                  _whole(tu3), _whole(ti3),
                  _whole(w2p), _whole(b2), _whole(w3p), _whole(b3p),
                  _whole(wfin), _whole(bf)],
        out_specs=pl.BlockSpec((1, 1, tile_b), lambda i: (i, 0, 0)),
        scratch_shapes=[pltpu.VMEM((chunk, W), jnp.float32)
                        for _ in range(2 * (tile_b // chunk))],
        compiler_params=pltpu.CompilerParams(
            dimension_semantics=("parallel",),
            vmem_limit_bytes=64 * 1024 * 1024,
        ),
    )(uidx, iidx, tu3, ti3, w2p, b2, w3p, b3p, wfin, bf)
    return out.reshape(b_pad)[:B].reshape(B, 1)


def kernel(user_idx, item_idx, user_emb, item_emb, w1, b1, w2, b2, w3, b3, wf, bf):
    return _forward(user_idx, item_idx, user_emb, item_emb,
                    w1, b1, w2, b2, w3, b3, wf, bf)


# speedup vs baseline: 1.5458x; 1.0394x over previous
import functools

import jax
import jax.numpy as jnp
from jax.experimental import pallas as pl
from jax.experimental.pallas import tpu as pltpu


def _round_up(x: int, m: int) -> int:
    return ((x + m - 1) // m) * m


def _make_transform_kernel(mf_dim: int):
    def _transform(tabT_ref, w_ref, b_ref, d_ref, out_ref):
        f32 = jnp.float32
        dn = (((0,), (0,)), ((), ()))
        gm = jax.lax.dot_general(tabT_ref[:mf_dim, :], d_ref[...], dn,
                                 preferred_element_type=f32)
        ml = jax.lax.dot_general(tabT_ref[mf_dim:, :], w_ref[...], dn,
                                 preferred_element_type=f32) + b_ref[...]
        out_ref[...] = jnp.concatenate([gm, ml], axis=1)
    return _transform


def _transform_table(tabT, w, b_row, diag, *, mf_dim: int,
                     row_tile: int = 2048):
    width, n = tabT.shape
    n_pad = _round_up(n, row_tile)
    tabT_p = jnp.pad(tabT, ((0, 0), (0, n_pad - n)))
    out = pl.pallas_call(
        _make_transform_kernel(mf_dim),
        out_shape=jax.ShapeDtypeStruct((n_pad, width), jnp.float32),
        grid=(n_pad // row_tile,),
        in_specs=[
            pl.BlockSpec((width, row_tile), lambda i: (0, i)),
            pl.BlockSpec(w.shape, lambda i: (0, 0)),
            pl.BlockSpec(b_row.shape, lambda i: (0, 0)),
            pl.BlockSpec(diag.shape, lambda i: (0, 0)),
        ],
        out_specs=pl.BlockSpec((row_tile, width), lambda i: (i, 0)),
        compiler_params=pltpu.CompilerParams(
            dimension_semantics=("parallel",),
            fuse_transposed_lhs_in_matmul=True),
    )(tabT_p, w, b_row, diag)
    return out


def _make_main_kernel(tile_b: int, chunk: int):
    n_chunks = tile_b // chunk

    def _main(uidx_ref, iidx_ref,
              tu_ref, ti_ref,
              w2p_ref, b2_ref,
              w3p_ref, b3p_ref,
              wfin_ref,
              bf_ref,
              out_ref,
              *scratch):
        f32 = jnp.float32
        for c in range(n_chunks):
            au_ref = scratch[2 * c]
            ai_ref = scratch[2 * c + 1]
            base = c * chunk
            for r in range(chunk):
                au_ref[r, :] = tu_ref[uidx_ref[0, 0, base + r], 0]
                ai_ref[r, :] = ti_ref[iidx_ref[0, 0, base + r], 0]
            a = au_ref[...]
            b = ai_ref[...]
            h = jnp.maximum(a + b, 0.0)
            h = jnp.maximum(
                jnp.dot(h, w2p_ref[...], preferred_element_type=f32)
                + b2_ref[...], 0.0)
            h = jnp.maximum(
                jnp.dot(h, w3p_ref[...], preferred_element_type=f32)
                + b3p_ref[...], 0.0)
            s = h + jnp.pad(a * b, ((0, 0), (0, h.shape[1] - a.shape[1])))
            dn = (((0,), (1,)), ((), ()))
            score_row = jax.lax.dot_general(
                wfin_ref[...], s, dn, preferred_element_type=f32) + bf_ref[...]
            out_ref[0, 0, pl.ds(base, chunk)] = jax.nn.sigmoid(
                score_row)[0, :]
    return _main


@functools.partial(jax.jit, static_argnames=("tile_b", "chunk"))
def _forward(user_idx, item_idx, user_emb, item_emb,
             w1, b1, w2, b2, w3, b3, wf, bf, *,
             tile_b: int = 8192, chunk: int = 2048):
    B = int(user_idx.shape[0])
    U, W = user_emb.shape
    half = w1.shape[0] // 2
    mf_dim = W - half
    l2 = w2.shape[1]

    diag_u = jnp.diag(wf[:mf_dim, 0])
    diag_i = jnp.eye(mf_dim, dtype=jnp.float32)
    zero_b = jnp.zeros_like(b1)
    tu = _transform_table(user_emb.T, w1[:half, :], zero_b, diag_u,
                          mf_dim=mf_dim)
    ti = _transform_table(item_emb.T, w1[half:, :], b1, diag_i,
                          mf_dim=mf_dim)
    tu3 = tu.reshape(tu.shape[0], 1, W)
    ti3 = ti.reshape(ti.shape[0], 1, W)

    w2p = jnp.concatenate([jnp.zeros((mf_dim, l2), jnp.float32), w2], axis=0)
    w3p = jnp.pad(w3, ((0, 0), (W, 0)))
    b3p = jnp.pad(b3, ((0, 0), (W, 0)))
    wfin = jnp.concatenate([wf[:mf_dim, :],
                            jnp.zeros((half, 1), jnp.float32),
                            wf[mf_dim:, :]], axis=0)

    b_pad = _round_up(B, tile_b)
    pad = b_pad - B
    uidx = jnp.pad(user_idx.astype(jnp.int32), (0, pad)).reshape(-1, 1, tile_b)
    iidx = jnp.pad(item_idx.astype(jnp.int32), (0, pad)).reshape(-1, 1, tile_b)
    num_tiles = b_pad // tile_b

    idx_spec = pl.BlockSpec((1, 1, tile_b), lambda i: (i, 0, 0),
                            memory_space=pltpu.SMEM)

    def _whole(a):
        return pl.BlockSpec(a.shape, lambda i: (0,) * a.ndim)

    out = pl.pallas_call(
        _make_main_kernel(tile_b, chunk),
        out_shape=jax.ShapeDtypeStruct((num_tiles, 1, tile_b), jnp.float32),
        grid=(num_tiles,),
        in_specs=[idx_spec, idx_spec,
                  _whole(tu3), _whole(ti3),
                  _whole(w2p), _whole(b2), _whole(w3p), _whole(b3p),
                  _whole(wfin), _whole(bf)],
        out_specs=pl.BlockSpec((1, 1, tile_b), lambda i: (i, 0, 0)),
        scratch_shapes=[pltpu.VMEM((chunk, W), jnp.float32)
                        for _ in range(2 * (tile_b // chunk))],
        compiler_params=pltpu.CompilerParams(
            dimension_semantics=("parallel",),
            vmem_limit_bytes=64 * 1024 * 1024,
        ),
    )(uidx, iidx, tu3, ti3, w2p, b2, w3p, b3p, wfin, bf)
    return out.reshape(b_pad)[:B].reshape(B, 1)


def kernel(user_idx, item_idx, user_emb, item_emb, w1, b1, w2, b2, w3, b3, wf, bf):
    return _forward(user_idx, item_idx, user_emb, item_emb,
                    w1, b1, w2, b2, w3, b3, wf, bf)


# speedup vs baseline: 1.5528x; 1.0045x over previous
import functools

import jax
import jax.numpy as jnp
from jax.experimental import pallas as pl
from jax.experimental.pallas import tpu as pltpu


def _round_up(x: int, m: int) -> int:
    return ((x + m - 1) // m) * m


def _make_transform_kernel(mf_dim: int):
    def _transform(tabT_ref, w_ref, b_ref, d_ref, out_ref):
        f32 = jnp.float32
        dn = (((0,), (0,)), ((), ()))
        gm = jax.lax.dot_general(tabT_ref[:mf_dim, :], d_ref[...], dn,
                                 preferred_element_type=f32)
        ml = jax.lax.dot_general(tabT_ref[mf_dim:, :], w_ref[...], dn,
                                 preferred_element_type=f32) + b_ref[...]
        out_ref[...] = jnp.concatenate([gm, ml], axis=1)
    return _transform


def _transform_table(tabT, w, b_row, diag, *, mf_dim: int,
                     row_tile: int = 4096):
    width, n = tabT.shape
    n_pad = _round_up(n, row_tile)
    tabT_p = jnp.pad(tabT, ((0, 0), (0, n_pad - n)))
    out = pl.pallas_call(
        _make_transform_kernel(mf_dim),
        out_shape=jax.ShapeDtypeStruct((n_pad, width), jnp.float32),
        grid=(n_pad // row_tile,),
        in_specs=[
            pl.BlockSpec((width, row_tile), lambda i: (0, i)),
            pl.BlockSpec(w.shape, lambda i: (0, 0)),
            pl.BlockSpec(b_row.shape, lambda i: (0, 0)),
            pl.BlockSpec(diag.shape, lambda i: (0, 0)),
        ],
        out_specs=pl.BlockSpec((row_tile, width), lambda i: (i, 0)),
        compiler_params=pltpu.CompilerParams(
            dimension_semantics=("parallel",),
            fuse_transposed_lhs_in_matmul=True),
    )(tabT_p, w, b_row, diag)
    return out


def _make_main_kernel(tile_b: int, chunk: int):
    n_chunks = tile_b // chunk

    def _main(uidx_ref, iidx_ref,
              tu_ref, ti_ref,
              w2p_ref, b2_ref,
              w3p_ref, b3p_ref,
              wfin_ref,
              bf_ref,
              out_ref,
              *scratch):
        f32 = jnp.float32
        for c in range(n_chunks):
            au_ref = scratch[2 * c]
            ai_ref = scratch[2 * c + 1]
            base = c * chunk
            for r in range(chunk):
                au_ref[r, :] = tu_ref[uidx_ref[0, 0, base + r], 0]
                ai_ref[r, :] = ti_ref[iidx_ref[0, 0, base + r], 0]
            a = au_ref[...]
            b = ai_ref[...]
            h = jnp.maximum(a + b, 0.0)
            h = jnp.maximum(
                jnp.dot(h, w2p_ref[...], preferred_element_type=f32)
                + b2_ref[...], 0.0)
            h = jnp.maximum(
                jnp.dot(h, w3p_ref[...], preferred_element_type=f32)
                + b3p_ref[...], 0.0)
            s = h + jnp.pad(a * b, ((0, 0), (0, h.shape[1] - a.shape[1])))
            dn = (((0,), (1,)), ((), ()))
            score_row = jax.lax.dot_general(
                wfin_ref[...], s, dn, preferred_element_type=f32) + bf_ref[...]
            out_ref[0, 0, pl.ds(base, chunk)] = jax.nn.sigmoid(
                score_row)[0, :]
    return _main


@functools.partial(jax.jit, static_argnames=("tile_b", "chunk"))
def _forward(user_idx, item_idx, user_emb, item_emb,
             w1, b1, w2, b2, w3, b3, wf, bf, *,
             tile_b: int = 8192, chunk: int = 2048):
    B = int(user_idx.shape[0])
    U, W = user_emb.shape
    half = w1.shape[0] // 2
    mf_dim = W - half
    l2 = w2.shape[1]

    diag_u = jnp.diag(wf[:mf_dim, 0])
    diag_i = jnp.eye(mf_dim, dtype=jnp.float32)
    zero_b = jnp.zeros_like(b1)
    tu = _transform_table(user_emb.T, w1[:half, :], zero_b, diag_u,
                          mf_dim=mf_dim)
    ti = _transform_table(item_emb.T, w1[half:, :], b1, diag_i,
                          mf_dim=mf_dim)
    tu3 = tu.reshape(tu.shape[0], 1, W)
    ti3 = ti.reshape(ti.shape[0], 1, W)

    w2p = jnp.concatenate([jnp.zeros((mf_dim, l2), jnp.float32), w2], axis=0)
    w3p = jnp.pad(w3, ((0, 0), (W, 0)))
    b3p = jnp.pad(b3, ((0, 0), (W, 0)))
    wfin = jnp.concatenate([wf[:mf_dim, :],
                            jnp.zeros((half, 1), jnp.float32),
                            wf[mf_dim:, :]], axis=0)

    b_pad = _round_up(B, tile_b)
    pad = b_pad - B
    uidx = jnp.pad(user_idx.astype(jnp.int32), (0, pad)).reshape(-1, 1, tile_b)
    iidx = jnp.pad(item_idx.astype(jnp.int32), (0, pad)).reshape(-1, 1, tile_b)
    num_tiles = b_pad // tile_b

    idx_spec = pl.BlockSpec((1, 1, tile_b), lambda i: (i, 0, 0),
                            memory_space=pltpu.SMEM)

    def _whole(a):
        return pl.BlockSpec(a.shape, lambda i: (0,) * a.ndim)

    out = pl.pallas_call(
        _make_main_kernel(tile_b, chunk),
        out_shape=jax.ShapeDtypeStruct((num_tiles, 1, tile_b), jnp.float32),
        grid=(num_tiles,),
        in_specs=[idx_spec, idx_spec,
                  _whole(tu3), _whole(ti3),
                  _whole(w2p), _whole(b2), _whole(w3p), _whole(b3p),
                  _whole(wfin), _whole(bf)],
        out_specs=pl.BlockSpec((1, 1, tile_b), lambda i: (i, 0, 0)),
        scratch_shapes=[pltpu.VMEM((chunk, W), jnp.float32)
                        for _ in range(2 * (tile_b // chunk))],
        compiler_params=pltpu.CompilerParams(
            dimension_semantics=("parallel",),
            vmem_limit_bytes=64 * 1024 * 1024,
        ),
    )(uidx, iidx, tu3, ti3, w2p, b2, w3p, b3p, wfin, bf)
    return out.reshape(b_pad)[:B].reshape(B, 1)


def kernel(user_idx, item_idx, user_emb, item_emb, w1, b1, w2, b2, w3, b3, wf, bf):
    return _forward(user_idx, item_idx, user_emb, item_emb,
                    w1, b1, w2, b2, w3, b3, wf, bf)


# speedup vs baseline: 1.5546x; 1.0012x over previous
import functools

import jax
import jax.numpy as jnp
from jax.experimental import pallas as pl
from jax.experimental.pallas import tpu as pltpu


def _round_up(x: int, m: int) -> int:
    return ((x + m - 1) // m) * m


def _make_transform_kernel(mf_dim: int):
    def _transform(tabT_ref, w_ref, b_ref, d_ref, out_ref):
        f32 = jnp.float32
        dn = (((0,), (0,)), ((), ()))
        gm = jax.lax.dot_general(tabT_ref[:mf_dim, :], d_ref[...], dn,
                                 preferred_element_type=f32)
        ml = jax.lax.dot_general(tabT_ref[mf_dim:, :], w_ref[...], dn,
                                 preferred_element_type=f32) + b_ref[...]
        out_ref[...] = jnp.concatenate([gm, ml], axis=1)
    return _transform


def _transform_table(tabT, w, b_row, diag, *, mf_dim: int,
                     row_tile: int = 4096):
    width, n = tabT.shape
    n_pad = _round_up(n, row_tile)
    tabT_p = jnp.pad(tabT, ((0, 0), (0, n_pad - n)))
    out = pl.pallas_call(
        _make_transform_kernel(mf_dim),
        out_shape=jax.ShapeDtypeStruct((n_pad, width), jnp.float32),
        grid=(n_pad // row_tile,),
        in_specs=[
            pl.BlockSpec((width, row_tile), lambda i: (0, i)),
            pl.BlockSpec(w.shape, lambda i: (0, 0)),
            pl.BlockSpec(b_row.shape, lambda i: (0, 0)),
            pl.BlockSpec(diag.shape, lambda i: (0, 0)),
        ],
        out_specs=pl.BlockSpec((row_tile, width), lambda i: (i, 0)),
        compiler_params=pltpu.CompilerParams(
            dimension_semantics=("parallel",),
            fuse_transposed_lhs_in_matmul=True),
    )(tabT_p, w, b_row, diag)
    return out


def _make_main_kernel(tile_b: int, chunk: int):
    n_chunks = tile_b // chunk

    def _main(uidx_ref, iidx_ref,
              tu_ref, ti_ref,
              w2p_ref, b2_ref,
              w3p_ref, b3p_ref,
              wfin_ref,
              bf_ref,
              out_ref,
              *scratch):
        f32 = jnp.float32
        for c in range(n_chunks):
            au_ref = scratch[2 * c]
            ai_ref = scratch[2 * c + 1]
            base = c * chunk
            for r in range(chunk):
                au_ref[r, :] = tu_ref[uidx_ref[0, 0, base + r], 0]
                ai_ref[r, :] = ti_ref[iidx_ref[0, 0, base + r], 0]
            a = au_ref[...]
            b = ai_ref[...]
            h = jnp.maximum(a + b, 0.0)
            h = jnp.maximum(
                jnp.dot(h, w2p_ref[...], preferred_element_type=f32)
                + b2_ref[...], 0.0)
            h = jnp.maximum(
                jnp.dot(h, w3p_ref[...], preferred_element_type=f32)
                + b3p_ref[...], 0.0)
            s = h + jnp.pad(a * b, ((0, 0), (0, h.shape[1] - a.shape[1])))
            dn = (((0,), (1,)), ((), ()))
            score_row = jax.lax.dot_general(
                wfin_ref[...], s, dn, preferred_element_type=f32) + bf_ref[...]
            out_ref[0, 0, pl.ds(base, chunk)] = jax.nn.sigmoid(
                score_row)[0, :]
    return _main


@functools.partial(jax.jit, static_argnames=("tile_b", "chunk"))
def _forward(user_idx, item_idx, user_emb, item_emb,
             w1, b1, w2, b2, w3, b3, wf, bf, *,
             tile_b: int = 8192, chunk: int = 2048):
    B = int(user_idx.shape[0])
    U, W = user_emb.shape
    half = w1.shape[0] // 2
    mf_dim = W - half
    l2 = w2.shape[1]

    diag_eye = jnp.eye(mf_dim, dtype=jnp.float32)
    zero_b = jnp.zeros_like(b1)
    tu = _transform_table(user_emb.T, w1[:half, :], zero_b, diag_eye,
                          mf_dim=mf_dim)
    ti = _transform_table(item_emb.T, w1[half:, :], b1, diag_eye,
                          mf_dim=mf_dim)
    tu3 = tu.reshape(tu.shape[0], 1, W)
    ti3 = ti.reshape(ti.shape[0], 1, W)

    w2p = jnp.concatenate([jnp.zeros((mf_dim, l2), jnp.float32), w2], axis=0)
    w3p = jnp.pad(w3, ((0, 0), (W, 0)))
    b3p = jnp.pad(b3, ((0, 0), (W, 0)))
    wfin = jnp.concatenate([wf[:mf_dim, :],
                            jnp.zeros((half, 1), jnp.float32),
                            wf[mf_dim:, :]], axis=0)

    b_pad = _round_up(B, tile_b)
    pad = b_pad - B
    uidx = jnp.pad(user_idx.astype(jnp.int32), (0, pad)).reshape(-1, 1, tile_b)
    iidx = jnp.pad(item_idx.astype(jnp.int32), (0, pad)).reshape(-1, 1, tile_b)
    num_tiles = b_pad // tile_b

    idx_spec = pl.BlockSpec((1, 1, tile_b), lambda i: (i, 0, 0),
                            memory_space=pltpu.SMEM)

    def _whole(a):
        return pl.BlockSpec(a.shape, lambda i: (0,) * a.ndim)

    out = pl.pallas_call(
        _make_main_kernel(tile_b, chunk),
        out_shape=jax.ShapeDtypeStruct((num_tiles, 1, tile_b), jnp.float32),
        grid=(num_tiles,),
        in_specs=[idx_spec, idx_spec,
                  _whole(tu3), _whole(ti3),
                  _whole(w2p), _whole(b2), _whole(w3p), _whole(b3p),
                  _whole(wfin), _whole(bf)],
        out_specs=pl.BlockSpec((1, 1, tile_b), lambda i: (i, 0, 0)),
        scratch_shapes=[pltpu.VMEM((chunk, W), jnp.float32)
                        for _ in range(2 * (tile_b // chunk))],
        compiler_params=pltpu.CompilerParams(
            dimension_semantics=("parallel",),
            vmem_limit_bytes=64 * 1024 * 1024,
        ),
    )(uidx, iidx, tu3, ti3, w2p, b2, w3p, b3p, wfin, bf)
    return out.reshape(b_pad)[:B].reshape(B, 1)


def kernel(user_idx, item_idx, user_emb, item_emb, w1, b1, w2, b2, w3, b3, wf, bf):
    return _forward(user_idx, item_idx, user_emb, item_emb,
                    w1, b1, w2, b2, w3, b3, wf, bf)
